# baseline (speedup 1.0000x reference)
import functools

import jax
import jax.numpy as jnp
from jax import lax
from jax.experimental import pallas as pl
from jax.experimental.pallas import tpu as pltpu
from jax.experimental.pallas import tpu_sc as plsc

_NSAMPLE = 2048
_SEM = 20
_B = 8
_N = 16384
_D = 128


def _score_body(label_ref, w_ref, g_ref, key_ref):
    lab = label_ref[...]
    w = w_ref[...]
    g = g_ref[...]
    sw = jnp.zeros((_B, _N), jnp.float32)
    for c in range(_SEM):
        m = lab == c
        cnt = jnp.sum(jnp.where(m, 1.0, 0.0), axis=1, keepdims=True)
        sw = sw + jnp.where(m, cnt, 0.0)
    sw = sw * w
    score = jnp.log(sw + 1e-12) + g
    bits = lax.bitcast_convert_type(score, jnp.uint32)
    flip = jnp.where(bits >= jnp.uint32(0x80000000),
                     jnp.uint32(0xFFFFFFFF), jnp.uint32(0x80000000))
    desc = (bits ^ flip) ^ jnp.uint32(0xFFFFFFFF)
    key_ref[...] = lax.bitcast_convert_type(desc, jnp.int32)


def _scores(seg_label, weights, g):
    return pl.pallas_call(
        _score_body,
        out_shape=jax.ShapeDtypeStruct((_B, _N), jnp.int32),
    )(seg_label.astype(jnp.int32), weights, g)


@functools.cache
def _build_sc_kernels():
    mesh = plsc.VectorSubcoreMesh(core_axis_name="c", subcore_axis_name="s")
    params = pltpu.CompilerParams(
        needs_layout_passes=False, use_tc_tiling_on_sc=False
    )
    sort_k = pl.kernel(
        _sort_body,
        mesh=mesh,
        compiler_params=params,
        out_type=jax.ShapeDtypeStruct((_B, _NSAMPLE), jnp.int32),
        scratch_types=[
            pltpu.VMEM((16400,), jnp.int32),
            pltpu.VMEM((16400,), jnp.int32),
            pltpu.VMEM((16400,), jnp.int32),
            pltpu.VMEM((16400,), jnp.int32),
            pltpu.VMEM((4096,), jnp.int32),
            pltpu.VMEM((4096,), jnp.int32),
            pltpu.VMEM((4096,), jnp.int32),
            pltpu.VMEM((4096,), jnp.int32),
            pltpu.VMEM((16400,), jnp.int32),
        ],
    )
    gather_k = pl.kernel(
        _gather_body,
        mesh=mesh,
        compiler_params=params,
        out_type=(
            jax.ShapeDtypeStruct((_B, 3 * _NSAMPLE), jnp.float32),
            jax.ShapeDtypeStruct((_B * _D * _NSAMPLE,), jnp.float32),
            jax.ShapeDtypeStruct((_B, _NSAMPLE), jnp.int32),
        ),
        scratch_types=[
            pltpu.VMEM((_NSAMPLE,), jnp.int32),
            pltpu.VMEM((2, _N), jnp.float32),
            pltpu.VMEM((2, _NSAMPLE), jnp.float32),
            pltpu.VMEM((3 * _N,), jnp.float32),
            pltpu.VMEM((3 * _NSAMPLE,), jnp.float32),
            pltpu.VMEM((_N,), jnp.int32),
            pltpu.VMEM((_NSAMPLE,), jnp.int32),
            pltpu.SemaphoreType.DMA,
            pltpu.SemaphoreType.DMA,
            pltpu.SemaphoreType.DMA,
            pltpu.SemaphoreType.DMA,
        ],
    )
    return sort_k, gather_k


def _sort_body(keys_hbm, idx_hbm, k0, k1, v0, v1, h0, h1, h2, h3, ordv):
    wid = lax.axis_index("s") * 2 + lax.axis_index("c")

    @pl.when(wid < _B)
    def _():
        b = wid
        lane = lax.iota(jnp.int32, 16)
        lane_base = lane * 1024
        ones = jnp.ones((16,), jnp.int32)
        zeros = jnp.zeros((16,), jnp.int32)

        pltpu.sync_copy(keys_hbm.at[b], k0.at[pl.ds(0, _N)])

        hsegs = (h0, h1, h2, h3)
        lane_pad = lane * 1025
        for p in range(4):
            kin = (k0, k1, k0, k1)[p]
            vin = (None, v1, v0, v1)[p]
            kout = (k1, k0, k1, k0)[p]
            vout = (v1, v0, v1, v0)[p]
            shift = 8 * p
            nbase = lane_base if p == 0 else lane_pad

            @plsc.parallel_loop(0, 256, unroll=2)
            def zero_body(t):
                for hst in hsegs:
                    hst[pl.ds(t * 16, 16)] = zeros

            def hist_body(q, _, kin=kin, shift=shift, nbase=nbase):
                for h, hst in enumerate(hsegs):
                    n = nbase + h * 256 + q
                    np_ = lane_pad + h * 256 + q
                    k = plsc.load_gather(kin, [n])
                    d = lax.shift_right_logical(k, shift) & 0xFF
                    addr = d * 16 + lane
                    c = plsc.load_gather(hst, [addr])
                    plsc.store_scatter(hst, [addr], c + 1)
                    plsc.store_scatter(ordv, [np_], c)
                return 0
            lax.fori_loop(0, 256, hist_body, 0)

            def scan_body(t, run):
                sl = pl.ds(t * 16, 16)
                va, vb, vc, vd = h0[sl], h1[sl], h2[sl], h3[sl]
                tot = va + vb + vc + vd
                s = plsc.cumsum(tot)
                base = s - tot + run
                h0[sl] = base
                h1[sl] = base + va
                h2[sl] = base + va + vb
                h3[sl] = base + va + vb + vc
                return run + jnp.sum(tot)
            lax.fori_loop(0, 256, scan_body, jnp.int32(0))

            def perm_body(q, kin=kin, vin=vin, kout=kout, vout=vout,
                          shift=shift, nbase=nbase, p=p):
                for h, hst in enumerate(hsegs):
                    n = nbase + h * 256 + q
                    np_ = lane_pad + h * 256 + q
                    k = plsc.load_gather(kin, [n])
                    v = (lane_base + h * 256 + q) if vin is None \
                        else plsc.load_gather(vin, [n])
                    d = lax.shift_right_logical(k, shift) & 0xFF
                    addr = d * 16 + lane
                    r = plsc.load_gather(hst, [addr]) + plsc.load_gather(ordv, [np_])
                    ofs = r if p == 3 else r + lax.shift_right_logical(r, 10)
                    plsc.store_scatter(kout, [ofs], k)
                    plsc.store_scatter(vout, [ofs], v)
            plsc.parallel_loop(0, 256, unroll=2)(perm_body)

        pltpu.sync_copy(v0.at[pl.ds(0, _NSAMPLE)], idx_hbm.at[b])


def _gather_body(xyz_hbm, feat_hbm, lab_hbm, idx_hbm,
                 oxyz, ofeat, olab,
                 idx_v, row_v, rout_v, xyz_v, xout_v, lab_v, lout_v,
                 sin0, sin1, sout0, sout1):
    wid = lax.axis_index("s") * 2 + lax.axis_index("c")
    b = wid // 4
    part = wid % 4
    d0 = part * 32
    lane = lax.iota(jnp.int32, 16)

    def fin(d):
        return feat_hbm.at[pl.ds((b * _D + d) * _N, _N)]

    def fout(d):
        return ofeat.at[pl.ds((b * _D + d) * _NSAMPLE, _NSAMPLE)]

    pltpu.make_async_copy(fin(d0), row_v.at[0], sin0).start()
    pltpu.make_async_copy(fin(d0 + 1), row_v.at[1], sin1).start()
    pltpu.sync_copy(idx_hbm.at[b], idx_v)

    def grp(jj, _):
        for u in range(2):
            d = d0 + jj * 2 + u
            sin = (sin0, sin1)[u]
            sout = (sout0, sout1)[u]
            rbuf = row_v.at[u]
            obuf = rout_v.at[u]
            pltpu.make_async_copy(fin(d), rbuf, sin).wait()

            @pl.when(jj > 0)
            def _():
                pltpu.make_async_copy(obuf, fout(d - 2), sout).wait()

            def g_body(t, _, rbuf=rbuf, obuf=obuf):
                iv = idx_v[pl.ds(t * 16, 16)]
                obuf[pl.ds(t * 16, 16)] = plsc.load_gather(rbuf, [iv])
                return 0
            lax.fori_loop(0, _NSAMPLE // 16, g_body, 0)
            pltpu.make_async_copy(obuf, fout(d), sout).start()

            @pl.when(jj < 15)
            def _():
                pltpu.make_async_copy(fin(d + 2), rbuf, sin).start()
        return 0
    lax.fori_loop(0, 16, grp, 0)
    pltpu.make_async_copy(rout_v.at[0], fout(d0 + 30), sout0).wait()
    pltpu.make_async_copy(rout_v.at[1], fout(d0 + 31), sout1).wait()

    @pl.when(part == 1)
    def _():
        pltpu.sync_copy(xyz_hbm.at[b], xyz_v)

        def x_body(t, _):
            jpos = t * 16 + lane
            iv = idx_v[pl.ds(t * 16, 16)]
            for r in range(3):
                vals = plsc.load_gather(xyz_v, [iv * 3 + r])
                plsc.store_scatter(xout_v, [jpos * 3 + r], vals)
            return 0
        lax.fori_loop(0, _NSAMPLE // 16, x_body, 0)
        pltpu.sync_copy(xout_v, oxyz.at[b])

    @pl.when(part == 2)
    def _():
        pltpu.sync_copy(lab_hbm.at[b], lab_v)

        def l_body(t, _):
            iv = idx_v[pl.ds(t * 16, 16)]
            lout_v[pl.ds(t * 16, 16)] = plsc.load_gather(lab_v, [iv])
            return 0
        lax.fori_loop(0, _NSAMPLE // 16, l_body, 0)
        pltpu.sync_copy(lout_v, olab.at[b])


def kernel(xyz, seg_feature, seg_label, weights):
    B, N, C = xyz.shape
    g = jax.random.gumbel(jax.random.key(42), (B, N), dtype=jnp.float32)
    lab32 = seg_label.astype(jnp.int32)
    keys = _scores(lab32, weights, g)
    sort_k, gather_k = _build_sc_kernels()
    idx = sort_k(keys)
    xyz_flat = jnp.reshape(xyz, (B, 3 * N))
    feat_flat = jnp.reshape(seg_feature, (B * _D * N,))
    oxyz, ofeat, olab = gather_k(xyz_flat, feat_flat, lab32, idx)
    sampled_xyz = jnp.reshape(oxyz, (B, _NSAMPLE, 3))
    sample_feat = jnp.reshape(ofeat, (B, _D, _NSAMPLE))
    return (sampled_xyz, sample_feat, olab.astype(seg_label.dtype))


# speedup vs baseline: 1.0416x; 1.0416x over previous
import functools

import jax
import jax.numpy as jnp
from jax import lax
from jax.experimental import pallas as pl
from jax.experimental.pallas import tpu as pltpu
from jax.experimental.pallas import tpu_sc as plsc

_NSAMPLE = 2048
_SEM = 20
_B = 8
_N = 16384
_D = 128


def _score_body(label_ref, w_ref, g_ref, key_ref):
    lab = label_ref[...]
    w = w_ref[...]
    g = g_ref[...]
    sw = jnp.zeros((_B, _N), jnp.float32)
    for c in range(_SEM):
        m = lab == c
        cnt = jnp.sum(jnp.where(m, 1.0, 0.0), axis=1, keepdims=True)
        sw = sw + jnp.where(m, cnt, 0.0)
    sw = sw * w
    score = jnp.log(sw + 1e-12) + g
    bits = lax.bitcast_convert_type(score, jnp.uint32)
    flip = jnp.where(bits >= jnp.uint32(0x80000000),
                     jnp.uint32(0xFFFFFFFF), jnp.uint32(0x80000000))
    desc = (bits ^ flip) ^ jnp.uint32(0xFFFFFFFF)
    key_ref[...] = lax.bitcast_convert_type(desc, jnp.int32)


def _scores(seg_label, weights, g):
    return pl.pallas_call(
        _score_body,
        out_shape=jax.ShapeDtypeStruct((_B, _N), jnp.int32),
    )(seg_label.astype(jnp.int32), weights, g)


@functools.cache
def _build_sc_kernels():
    mesh = plsc.VectorSubcoreMesh(core_axis_name="c", subcore_axis_name="s")
    params = pltpu.CompilerParams(
        needs_layout_passes=False, use_tc_tiling_on_sc=False
    )
    sort_k = pl.kernel(
        _sort_body,
        mesh=mesh,
        compiler_params=params,
        out_type=jax.ShapeDtypeStruct((_B, _NSAMPLE), jnp.int32),
        scratch_types=[
            pltpu.VMEM((16400,), jnp.int32),
            pltpu.VMEM((16400,), jnp.int32),
            pltpu.VMEM((16400,), jnp.int32),
            pltpu.VMEM((16400,), jnp.int32),
            pltpu.VMEM((4096,), jnp.int32),
            pltpu.VMEM((4096,), jnp.int32),
            pltpu.VMEM((4096,), jnp.int32),
            pltpu.VMEM((4096,), jnp.int32),
            pltpu.VMEM((16400,), jnp.int32),
            pltpu.VMEM((4096,), jnp.int32),
            pltpu.VMEM((256,), jnp.int32),
        ],
    )
    gather_k = pl.kernel(
        _gather_body,
        mesh=mesh,
        compiler_params=params,
        out_type=(
            jax.ShapeDtypeStruct((_B, 3 * _NSAMPLE), jnp.float32),
            jax.ShapeDtypeStruct((_B * _D * _NSAMPLE,), jnp.float32),
            jax.ShapeDtypeStruct((_B, _NSAMPLE), jnp.int32),
        ),
        scratch_types=[
            pltpu.VMEM((_NSAMPLE,), jnp.int32),
            pltpu.VMEM((2, 2 * _N), jnp.float32),
            pltpu.VMEM((2, 2 * _NSAMPLE), jnp.float32),
            pltpu.VMEM((3 * _NSAMPLE,), jnp.float32),
            pltpu.VMEM((_N,), jnp.int32),
            pltpu.VMEM((_NSAMPLE,), jnp.int32),
            pltpu.SemaphoreType.DMA,
            pltpu.SemaphoreType.DMA,
            pltpu.SemaphoreType.DMA,
            pltpu.SemaphoreType.DMA,
        ],
    )
    return sort_k, gather_k


def _sort_body(keys_hbm, idx_hbm, k0, k1, v0, v1, h0, h1, h2, h3, ordv,
               stot, parray):
    wid = lax.axis_index("s") * 2 + lax.axis_index("c")

    @pl.when(wid < _B)
    def _():
        b = wid
        lane = lax.iota(jnp.int32, 16)
        lane_base = lane * 1024
        ones = jnp.ones((16,), jnp.int32)
        zeros = jnp.zeros((16,), jnp.int32)

        pltpu.sync_copy(keys_hbm.at[b], k0.at[pl.ds(0, _N)])

        hsegs = (h0, h1, h2, h3)
        lane_pad = lane * 1025
        for p in range(4):
            kin = (k0, k1, k0, k1)[p]
            vin = (None, v1, v0, v1)[p]
            kout = (k1, k0, k1, k0)[p]
            vout = (v1, v0, v1, v0)[p]
            shift = 8 * p
            nbase = lane_base if p == 0 else lane_pad

            @plsc.parallel_loop(0, 256, unroll=2)
            def zero_body(t):
                for hst in hsegs:
                    hst[pl.ds(t * 16, 16)] = zeros

            def hist_body(q, _, kin=kin, shift=shift, nbase=nbase):
                for h, hst in enumerate(hsegs):
                    n = nbase + h * 256 + q
                    np_ = lane_pad + h * 256 + q
                    k = plsc.load_gather(kin, [n])
                    d = lax.shift_right_logical(k, shift) & 0xFF
                    addr = d * 16 + lane
                    c = plsc.load_gather(hst, [addr])
                    plsc.store_scatter(hst, [addr], c + 1)
                    plsc.store_scatter(ordv, [np_], c)
                return 0
            lax.fori_loop(0, 256, hist_body, 0)

            @plsc.parallel_loop(0, 256, unroll=2)
            def scanA_body(t):
                sl = pl.ds(t * 16, 16)
                va, vb, vc, vd = h0[sl], h1[sl], h2[sl], h3[sl]
                tot = va + vb + vc + vd
                s = plsc.cumsum(tot)
                base = s - tot
                h0[sl] = base
                h1[sl] = base + va
                h2[sl] = base + va + vb
                h3[sl] = base + va + vb + vc
                stot[pl.ds(t * 16, 16)] = jnp.broadcast_to(jnp.sum(tot), (16,))

            def scanB_body(tt, run):
                t = tt * 16 + lane
                tv = plsc.load_gather(stot, [t * 16 + lane])
                s2 = plsc.cumsum(tv)
                parray[pl.ds(tt * 16, 16)] = s2 - tv + run
                return run + jnp.sum(tv)
            lax.fori_loop(0, 16, scanB_body, jnp.int32(0))

            def perm_body(q, kin=kin, vin=vin, kout=kout, vout=vout,
                          shift=shift, nbase=nbase, p=p):
                for h, hst in enumerate(hsegs):
                    n = nbase + h * 256 + q
                    np_ = lane_pad + h * 256 + q
                    k = plsc.load_gather(kin, [n])
                    v = (lane_base + h * 256 + q) if vin is None \
                        else plsc.load_gather(vin, [n])
                    d = lax.shift_right_logical(k, shift) & 0xFF
                    addr = d * 16 + lane
                    r = (plsc.load_gather(hst, [addr])
                         + plsc.load_gather(ordv, [np_])
                         + plsc.load_gather(parray, [d]))
                    ofs = r if p == 3 else r + lax.shift_right_logical(r, 10)
                    plsc.store_scatter(kout, [ofs], k)
                    plsc.store_scatter(vout, [ofs], v)
            plsc.parallel_loop(0, 256, unroll=2)(perm_body)

        pltpu.sync_copy(v0.at[pl.ds(0, _NSAMPLE)], idx_hbm.at[b])


def _gather_body(xyz_hbm, feat_hbm, lab_hbm, idx_hbm,
                 oxyz, ofeat, olab,
                 idx_v, row_v, rout_v, xout_v, lab_v, lout_v,
                 sin0, sin1, sout0, sout1):
    wid = lax.axis_index("s") * 2 + lax.axis_index("c")
    b = wid // 4
    part = wid % 4
    d0 = part * 32
    lane = lax.iota(jnp.int32, 16)

    def fin(d):
        return feat_hbm.at[pl.ds((b * _D + d) * _N, 2 * _N)]

    def fout(d):
        return ofeat.at[pl.ds((b * _D + d) * _NSAMPLE, 2 * _NSAMPLE)]

    pltpu.make_async_copy(fin(d0), row_v.at[0], sin0).start()
    pltpu.make_async_copy(fin(d0 + 2), row_v.at[1], sin1).start()
    pltpu.sync_copy(idx_hbm.at[b], idx_v)

    def grp(jj, _):
        for u in range(2):
            d = d0 + (jj * 2 + u) * 2
            sin = (sin0, sin1)[u]
            sout = (sout0, sout1)[u]
            rbuf = row_v.at[u]
            obuf = rout_v.at[u]
            pltpu.make_async_copy(fin(d), rbuf, sin).wait()

            @pl.when(jj > 0)
            def _():
                pltpu.make_async_copy(obuf, fout(d - 4), sout).wait()

            def g_body(t, _, rbuf=rbuf, obuf=obuf):
                iv = idx_v[pl.ds(t * 16, 16)]
                obuf[pl.ds(t * 16, 16)] = plsc.load_gather(rbuf, [iv])
                obuf[pl.ds(_NSAMPLE + t * 16, 16)] = plsc.load_gather(
                    rbuf, [iv + _N])
                return 0
            lax.fori_loop(0, _NSAMPLE // 16, g_body, 0)
            pltpu.make_async_copy(obuf, fout(d), sout).start()

            @pl.when(jj < 7)
            def _():
                pltpu.make_async_copy(fin(d + 4), rbuf, sin).start()
        return 0
    lax.fori_loop(0, 8, grp, 0)
    pltpu.make_async_copy(rout_v.at[0], fout(d0 + 28), sout0).wait()
    pltpu.make_async_copy(rout_v.at[1], fout(d0 + 30), sout1).wait()

    @pl.when(part == 1)
    def _():
        pltpu.sync_copy(xyz_hbm.at[b, pl.ds(0, 2 * _N)], row_v.at[0])
        pltpu.sync_copy(xyz_hbm.at[b, pl.ds(2 * _N, _N)],
                        row_v.at[1, pl.ds(0, _N)])

        def x_body(t, _):
            jpos = t * 16 + lane
            iv = idx_v[pl.ds(t * 16, 16)]
            for r in range(3):
                f = iv * 3 + r
                vals = plsc.load_gather(
                    row_v, [lax.shift_right_logical(f, 15), f & (2 * _N - 1)])
                plsc.store_scatter(xout_v, [jpos * 3 + r], vals)
            return 0
        lax.fori_loop(0, _NSAMPLE // 16, x_body, 0)
        pltpu.sync_copy(xout_v, oxyz.at[b])

    @pl.when(part == 2)
    def _():
        pltpu.sync_copy(lab_hbm.at[b], lab_v)

        def l_body(t, _):
            iv = idx_v[pl.ds(t * 16, 16)]
            lout_v[pl.ds(t * 16, 16)] = plsc.load_gather(lab_v, [iv])
            return 0
        lax.fori_loop(0, _NSAMPLE // 16, l_body, 0)
        pltpu.sync_copy(lout_v, olab.at[b])


def kernel(xyz, seg_feature, seg_label, weights):
    B, N, C = xyz.shape
    g = jax.random.gumbel(jax.random.key(42), (B, N), dtype=jnp.float32)
    lab32 = seg_label.astype(jnp.int32)
    keys = _scores(lab32, weights, g)
    sort_k, gather_k = _build_sc_kernels()
    idx = sort_k(keys)
    xyz_flat = jnp.reshape(xyz, (B, 3 * N))
    feat_flat = jnp.reshape(seg_feature, (B * _D * N,))
    oxyz, ofeat, olab = gather_k(xyz_flat, feat_flat, lab32, idx)
    sampled_xyz = jnp.reshape(oxyz, (B, _NSAMPLE, 3))
    sample_feat = jnp.reshape(ofeat, (B, _D, _NSAMPLE))
    return (sampled_xyz, sample_feat, olab.astype(seg_label.dtype))
